# baseline (speedup 1.0000x reference)
import functools

import jax
import jax.numpy as jnp
from jax import lax
from jax.experimental import pallas as pl
from jax.experimental.pallas import tpu as pltpu
from jax.experimental.pallas import tpu_sc as plsc

_NUM_LUT = 3
_NV = 128
_OUT_F = 2048
_LUT = 16
_VEC = 16
_VPG = 8
_NG = 16
_CHUNK = 1024
_VPW = 4


def _sc_gather(gate_hbm, cbf_hbm, out_hbm, gvmem, ovmem, cbvmem):
    wid = lax.axis_index("s") * 2 + lax.axis_index("c")
    v0 = wid * _VPW
    for l in range(_NUM_LUT):
        for i in range(_VPW):
            pltpu.sync_copy(
                cbf_hbm.at[pl.ds((l * _NV + v0 + i) * 256, 256)],
                cbvmem.at[pl.ds((l * _VPW + i) * 256, 256)])
    iota = lax.iota(jnp.int32, 16)
    perms = [iota ^ s for s in (1, 2, 4, 8)]
    for i in range(_VPW):
        v = v0 + i
        for c in range(_OUT_F // _CHUNK):
            o0 = c * _CHUNK
            for l in range(_NUM_LUT):
                pltpu.sync_copy(
                    gate_hbm.at[pl.ds(((l * _NV + v) * _OUT_F + o0) * _VEC,
                                      _CHUNK * _VEC)],
                    gvmem.at[pl.ds(l * _CHUNK * _VEC, _CHUNK * _VEC)])

            def body(o, carry):
                acc = jnp.zeros((_VEC,), jnp.float32)
                for l in range(_NUM_LUT):
                    g = gvmem[pl.ds(l * _CHUNK * _VEC + o * _VEC, _VEC)]
                    m = lax.bitcast_convert_type(
                        (lax.bitcast_convert_type(g, jnp.int32) & ~15)
                        | (15 - iota), jnp.float32)
                    for p in perms:
                        m = jnp.maximum(
                            m, m.at[p].get(mode="promise_in_bounds"))
                    idx = 15 - (lax.bitcast_convert_type(m, jnp.int32) & 15)
                    acc = acc + plsc.load_gather(
                        cbvmem,
                        [(l * _VPW + i) * 256 + idx * _VEC + iota])
                ovmem[pl.ds(o * _VEC, _VEC)] = acc
                return carry

            lax.fori_loop(0, _CHUNK, body, 0)
            pltpu.sync_copy(
                ovmem,
                out_hbm.at[pl.ds((v * _OUT_F + o0) * _VEC, _CHUNK * _VEC)])


def _affine_body(w_ref, sc_ref, zr_ref, out_ref):
    w = jnp.concatenate([w_ref[0, vv] for vv in range(_VPG)], axis=1)
    s = jax.lax.broadcast_in_dim(sc_ref[0], (_OUT_F, 128), (0, 1))
    z = jax.lax.broadcast_in_dim(zr_ref[0], (_OUT_F, 128), (0, 1))
    out_ref[...] = (w - z) * s


@jax.jit
def kernel(gate, codebook, scales, zeros):
    mesh = plsc.VectorSubcoreMesh(core_axis_name="c", subcore_axis_name="s")
    wsum = pl.kernel(
        _sc_gather,
        out_type=jax.ShapeDtypeStruct((_NV * _OUT_F * _VEC,), jnp.float32),
        mesh=mesh,
        compiler_params=pltpu.CompilerParams(needs_layout_passes=False),
        scratch_types=[
            pltpu.VMEM((_NUM_LUT * _CHUNK * _VEC,), jnp.float32),
            pltpu.VMEM((_CHUNK * _VEC,), jnp.float32),
            pltpu.VMEM((_NUM_LUT * _VPW * 256,), jnp.float32),
        ],
    )(gate.reshape(-1), codebook.reshape(-1))

    wv = wsum.reshape(_NG, _VPG, _OUT_F, _VEC)
    st = scales.T.reshape(_NG, _OUT_F, 1)
    zt = zeros.astype(jnp.float32).T.reshape(_NG, _OUT_F, 1)
    return pl.pallas_call(
        _affine_body,
        grid=(_NG,),
        in_specs=[
            pl.BlockSpec((1, _VPG, _OUT_F, _VEC), lambda g: (g, 0, 0, 0)),
            pl.BlockSpec((1, _OUT_F, 1), lambda g: (g, 0, 0)),
            pl.BlockSpec((1, _OUT_F, 1), lambda g: (g, 0, 0)),
        ],
        out_specs=pl.BlockSpec((_OUT_F, 128), lambda g: (0, g)),
        out_shape=jax.ShapeDtypeStruct((_OUT_F, _NV * _VEC), jnp.float32),
    )(wv, st, zt)


# speedup vs baseline: 1.2989x; 1.2989x over previous
import functools

import jax
import jax.numpy as jnp
from jax import lax
from jax.experimental import pallas as pl
from jax.experimental.pallas import tpu as pltpu
from jax.experimental.pallas import tpu_sc as plsc

_NUM_LUT = 3
_NV = 128
_OUT_F = 2048
_LUT = 16
_VEC = 16
_VPG = 8
_NG = 16
_R = _OUT_F // 8

_NG_TC = 14
_V_SC0 = _NG_TC * _VPG
_NV_SC = _NV - _V_SC0
_CHUNK = 1024


def _tc_body(gate_ref, cb_ref, sc_ref, zr_ref, out_ref):
    lane = jax.lax.broadcasted_iota(jnp.int32, (_R, 128), 1)
    seg = lane % _LUT
    inv16 = (15 - seg).astype(jnp.int16)
    low4 = jnp.int16(15)
    smasks = [seg < _LUT - s for s in (1, 2, 4, 8)]
    neg = jnp.bfloat16(-3.0e38)
    l3 = jax.lax.broadcasted_iota(jnp.int32, (384, 384), 0)
    c3 = jax.lax.broadcasted_iota(jnp.int32, (384, 384), 1)
    e3 = jnp.where((l3 % _LUT == 0) & (l3 // _LUT == c3 // _LUT),
                   1.0, 0.0).astype(jnp.bfloat16)
    inv48 = jnp.concatenate([15 - seg] * _NUM_LUT, axis=1)
    li = jax.lax.broadcasted_iota(jnp.int32, (128, 128), 0)
    ci = jax.lax.broadcasted_iota(jnp.int32, (128, 128), 1)
    bdmask = (li // _LUT) == (ci // _LUT)

    ai = jax.lax.broadcasted_iota(jnp.int32, (_VPG, 128), 0)
    cj = jax.lax.broadcasted_iota(jnp.int32, (_VPG, 128), 1)
    e8 = jnp.where(cj // _LUT == ai, 1.0, 0.0).astype(jnp.float32)
    s128 = jax.lax.dot(sc_ref[0], e8, precision=jax.lax.Precision.HIGHEST)
    z128 = jax.lax.dot(zr_ref[0], e8, precision=jax.lax.Precision.HIGHEST)

    for vv in range(8):
        xs = []
        bds = []
        for l in range(_NUM_LUT):
            gi = gate_ref[l, vv]
            xi = jax.lax.bitcast_convert_type(gi.astype(jnp.bfloat16), jnp.int16)
            x = jax.lax.bitcast_convert_type((xi & ~low4) | inv16, jnp.bfloat16)
            for i, s in enumerate((1, 2, 4, 8)):
                y = pltpu.roll(x, 128 - s, 1)
                x = jnp.maximum(x, jnp.where(smasks[i], y, neg))
            xs.append(x)
            bds.append(jnp.where(bdmask, jnp.tile(cb_ref[l, 0, vv], (8, 8)),
                                 jnp.bfloat16(0.0)))
        x3 = jnp.concatenate(xs, axis=1)
        m3 = jax.lax.dot(x3, e3, preferred_element_type=jnp.float32)
        wi = (jax.lax.bitcast_convert_type(m3, jnp.int32) >> 16) & 15
        oh = jnp.where(wi == inv48, 1.0, 0.0).astype(jnp.bfloat16)
        bd = jnp.concatenate(bds, axis=0)
        w = jax.lax.dot(oh, bd,
                        preferred_element_type=jnp.float32)
        out_ref[vv] = (w - z128) * s128


def _sc_gather(gate_hbm, cbf_hbm, out_hbm, gvmem, ovmem, cbvmem):
    wid = lax.axis_index("s") * 2 + lax.axis_index("c")
    v = _V_SC0 + wid // 2
    o0 = (wid % 2) * _CHUNK
    for l in range(_NUM_LUT):
        pltpu.sync_copy(cbf_hbm.at[pl.ds((l * _NV + v) * 256, 256)],
                        cbvmem.at[pl.ds(l * 256, 256)])
    iota = lax.iota(jnp.int32, 16)
    perms = [iota ^ s for s in (1, 2, 4, 8)]
    for l in range(_NUM_LUT):
        pltpu.sync_copy(
            gate_hbm.at[pl.ds(((l * _NV + v) * _OUT_F + o0) * _VEC,
                              _CHUNK * _VEC)],
            gvmem.at[pl.ds(l * _CHUNK * _VEC, _CHUNK * _VEC)])

    def body(o, carry):
        acc = jnp.zeros((_VEC,), jnp.float32)
        for l in range(_NUM_LUT):
            g = gvmem[pl.ds(l * _CHUNK * _VEC + o * _VEC, _VEC)]
            m = lax.bitcast_convert_type(
                (lax.bitcast_convert_type(g, jnp.int32) & ~15)
                | (15 - iota), jnp.float32)
            for p in perms:
                m = jnp.maximum(m, m.at[p].get(mode="promise_in_bounds"))
            idx = 15 - (lax.bitcast_convert_type(m, jnp.int32) & 15)
            acc = acc + plsc.load_gather(
                cbvmem, [l * 256 + idx * _VEC + iota])
        ovmem[pl.ds(o * _VEC, _VEC)] = acc
        return carry

    lax.fori_loop(0, _CHUNK, body, 0)
    pltpu.sync_copy(
        ovmem,
        out_hbm.at[pl.ds(((v - _V_SC0) * _OUT_F + o0) * _VEC, _CHUNK * _VEC)])


def _affine_body(w_ref, sc_ref, zr_ref, out_ref):
    w = jnp.concatenate([w_ref[0, vv] for vv in range(_VPG)], axis=1)
    s = jax.lax.broadcast_in_dim(sc_ref[0], (_OUT_F, 128), (0, 1))
    z = jax.lax.broadcast_in_dim(zr_ref[0], (_OUT_F, 128), (0, 1))
    out_ref[...] = (w - z) * s


@jax.jit
def kernel(gate, codebook, scales, zeros):
    gv = gate.reshape(_NUM_LUT, _NV, _R, 128)
    cb = codebook.reshape(_NUM_LUT, _NG, _VPG, _LUT, _VEC).astype(jnp.bfloat16)
    st = scales.T.reshape(_NG, _R, _VPG)
    zt = zeros.astype(jnp.float32).T.reshape(_NG, _R, _VPG)

    res_tc = pl.pallas_call(
        _tc_body,
        grid=(_NG_TC,),
        in_specs=[
            pl.BlockSpec((_NUM_LUT, _VPG, _R, 128), lambda g: (0, g, 0, 0)),
            pl.BlockSpec((_NUM_LUT, 1, _VPG, _LUT, _VEC),
                         lambda g: (0, g, 0, 0, 0)),
            pl.BlockSpec((1, _R, _VPG), lambda g: (g, 0, 0)),
            pl.BlockSpec((1, _R, _VPG), lambda g: (g, 0, 0)),
        ],
        out_specs=pl.BlockSpec((_VPG, _R, 128), lambda g: (g, 0, 0)),
        out_shape=jax.ShapeDtypeStruct((_NG_TC * _VPG, _R, 128), jnp.float32),
    )(gv, cb, st, zt)

    mesh = plsc.VectorSubcoreMesh(core_axis_name="c", subcore_axis_name="s")
    wsum = pl.kernel(
        _sc_gather,
        out_type=jax.ShapeDtypeStruct((_NV_SC * _OUT_F * _VEC,), jnp.float32),
        mesh=mesh,
        compiler_params=pltpu.CompilerParams(needs_layout_passes=False),
        scratch_types=[
            pltpu.VMEM((_NUM_LUT * _CHUNK * _VEC,), jnp.float32),
            pltpu.VMEM((_CHUNK * _VEC,), jnp.float32),
            pltpu.VMEM((_NUM_LUT * 256,), jnp.float32),
        ],
    )(gate.reshape(-1), codebook.reshape(-1))

    wv = wsum.reshape(_NG - _NG_TC, _VPG, _OUT_F, _VEC)
    stc = scales.T.reshape(_NG, _OUT_F, 1)
    ztc = zeros.astype(jnp.float32).T.reshape(_NG, _OUT_F, 1)
    out_sc = pl.pallas_call(
        _affine_body,
        grid=(_NG - _NG_TC,),
        in_specs=[
            pl.BlockSpec((1, _VPG, _OUT_F, _VEC), lambda g: (g, 0, 0, 0)),
            pl.BlockSpec((1, _OUT_F, 1), lambda g: (g + _NG_TC, 0, 0)),
            pl.BlockSpec((1, _OUT_F, 1), lambda g: (g + _NG_TC, 0, 0)),
        ],
        out_specs=pl.BlockSpec((_OUT_F, 128), lambda g: (0, g)),
        out_shape=jax.ShapeDtypeStruct((_OUT_F, (_NG - _NG_TC) * 128),
                                       jnp.float32),
    )(wv, stc, ztc)

    left = res_tc.reshape(_NG_TC * _VPG, _OUT_F, _VEC).transpose(1, 0, 2)
    left = left.reshape(_OUT_F, _NG_TC * 128)
    return jnp.concatenate([left, out_sc], axis=1)


# speedup vs baseline: 3.3731x; 2.5969x over previous
import functools

import jax
import jax.numpy as jnp
from jax.experimental import pallas as pl
from jax.experimental.pallas import tpu as pltpu

_NUM_LUT = 3
_NV = 128
_OUT_F = 2048
_LUT = 16
_VEC = 16
_VPG = 8
_NG = 16
_R = _OUT_F // 8


def _body(gate_ref, cb_ref, sc_ref, zr_ref, out_ref):
    lane = jax.lax.broadcasted_iota(jnp.int32, (_R, 128), 1)
    seg = lane % _LUT
    inv16 = (15 - seg).astype(jnp.int16)
    low4 = jnp.int16(15)
    smasks = [seg < _LUT - s for s in (1, 2, 4, 8)]
    neg = jnp.bfloat16(-3.0e38)
    l3 = jax.lax.broadcasted_iota(jnp.int32, (384, 384), 0)
    c3 = jax.lax.broadcasted_iota(jnp.int32, (384, 384), 1)
    e3 = jnp.where((l3 % _LUT == 0) & (l3 // _LUT == c3 // _LUT),
                   1.0, 0.0).astype(jnp.bfloat16)
    inv48 = jnp.concatenate([15 - seg] * _NUM_LUT, axis=1)
    li = jax.lax.broadcasted_iota(jnp.int32, (128, 128), 0)
    ci = jax.lax.broadcasted_iota(jnp.int32, (128, 128), 1)
    bdmask = (li // _LUT) == (ci // _LUT)

    ai = jax.lax.broadcasted_iota(jnp.int32, (_VPG, 128), 0)
    cj = jax.lax.broadcasted_iota(jnp.int32, (_VPG, 128), 1)
    e8 = jnp.where(cj // _LUT == ai, 1.0, 0.0).astype(jnp.float32)
    s128 = jax.lax.dot(sc_ref[0], e8, precision=jax.lax.Precision.HIGHEST)
    z128 = jax.lax.dot(zr_ref[0], e8, precision=jax.lax.Precision.HIGHEST)

    for vv in range(8):
        xs = []
        bds = []
        for l in range(_NUM_LUT):
            gi = gate_ref[l, vv]
            xi = jax.lax.bitcast_convert_type(gi.astype(jnp.bfloat16), jnp.int16)
            x = jax.lax.bitcast_convert_type((xi & ~low4) | inv16, jnp.bfloat16)
            for i, s in enumerate((1, 2, 4, 8)):
                y = pltpu.roll(x, 128 - s, 1)
                x = jnp.maximum(x, jnp.where(smasks[i], y, neg))
            xs.append(x)
            bds.append(jnp.where(bdmask, jnp.tile(cb_ref[l, 0, vv], (8, 8)),
                                 jnp.bfloat16(0.0)))
        x3 = jnp.concatenate(xs, axis=1)
        m3 = jax.lax.dot(x3, e3, preferred_element_type=jnp.float32)
        wi = (jax.lax.bitcast_convert_type(m3, jnp.int32) >> 16) & 15
        oh = jnp.where(wi == inv48, 1.0, 0.0).astype(jnp.bfloat16)
        bd = jnp.concatenate(bds, axis=0)
        w = jax.lax.dot(oh, bd,
                        preferred_element_type=jnp.float32)
        out_ref[vv] = (w - z128) * s128


@jax.jit
def kernel(gate, codebook, scales, zeros):
    gv = gate.reshape(_NUM_LUT, _NV, _R, 128)
    cb = codebook.reshape(_NUM_LUT, _NG, _VPG, _LUT, _VEC).astype(jnp.bfloat16)
    st = scales.T.reshape(_NG, _R, _VPG)
    zt = zeros.astype(jnp.float32).T.reshape(_NG, _R, _VPG)

    res = pl.pallas_call(
        _body,
        grid=(_NG,),
        in_specs=[
            pl.BlockSpec((_NUM_LUT, _VPG, _R, 128), lambda g: (0, g, 0, 0)),
            pl.BlockSpec((_NUM_LUT, 1, _VPG, _LUT, _VEC),
                         lambda g: (0, g, 0, 0, 0)),
            pl.BlockSpec((1, _R, _VPG), lambda g: (g, 0, 0)),
            pl.BlockSpec((1, _R, _VPG), lambda g: (g, 0, 0)),
        ],
        out_specs=pl.BlockSpec((_VPG, _R, 128), lambda g: (g, 0, 0)),
        out_shape=jax.ShapeDtypeStruct((_NV, _R, 128), jnp.float32),
    )(gv, cb, st, zt)

    return res.reshape(_NV, _OUT_F, _VEC).transpose(1, 0, 2).reshape(_OUT_F, _NV * _VEC)
